# baseline (speedup 1.0000x reference)
import jax
import jax.numpy as jnp
from jax.experimental import pallas as pl
from jax.experimental.pallas import tpu as pltpu

N = 100000
F = 128
K = 1024
BLK = 5000


def _score_body(x_ref, w_ref, o_ref):
    x = x_ref[...]
    w = w_ref[...]
    o_ref[...] = jnp.sum(x * w, axis=1, keepdims=True)


def kernel(node_embs, scorer):
    norm = jnp.sqrt(jnp.sum(scorer * scorer))
    wn = (scorer / norm).reshape(1, F)
    scores = pl.pallas_call(
        _score_body,
        grid=(N // BLK,),
        in_specs=[
            pl.BlockSpec((BLK, F), lambda i: (i, 0)),
            pl.BlockSpec((1, F), lambda i: (0, 0)),
        ],
        out_specs=pl.BlockSpec((BLK, 1), lambda i: (i, 0)),
        out_shape=jax.ShapeDtypeStruct((N, 1), jnp.float32),
    )(node_embs, wn)
    flat = scores.reshape(-1)
    vals, idx = jax.lax.top_k(flat, K)
    gathered = jnp.take(node_embs, idx, axis=0)
    out = gathered * jnp.tanh(vals)[:, None]
    return out.T


# speedup vs baseline: 3.3343x; 3.3343x over previous
import functools

import jax
import jax.numpy as jnp
from jax import lax
from jax.experimental import pallas as pl
from jax.experimental.pallas import tpu as pltpu
from jax.experimental.pallas import tpu_sc as plsc

N = 100000
F = 128
K = 1024
BLK = 6272
GRID = 16
NPAD = BLK * GRID
NW = 32
CHUNK = NPAD // NW
CAP = 128
INT_MIN = -2147483648
MASK31 = 0x7FFFFFFF


def _k1_body(norm_ref, x_ref, w_ref, keys_ref, thr_ref, skeys_ref):
    b = pl.program_id(0)
    s = lax.dot_general(
        w_ref[...].reshape(1, F), x_ref[...],
        dimension_numbers=(((1,), (1,)), ((), ())),
        preferred_element_type=jnp.float32)
    s = s / norm_ref[0, 0]
    k = lax.bitcast_convert_type(s, jnp.int32)
    k = jnp.where(k < 0, k ^ MASK31, k)
    pos = b * BLK + lax.broadcasted_iota(jnp.int32, (1, BLK), 1)
    k = jnp.where(pos < N, k, INT_MIN)
    keys_ref[...] = k[None]
    skeys_ref[pl.ds(b, 1), :] = k

    @pl.when(b == GRID - 1)
    def _():
        allk = skeys_ref[...]

        def body(i, t):
            cand = t + (jnp.int32(1) << (31 - i))
            cnt = jnp.sum((allk >= cand).astype(jnp.int32))
            return jnp.where(cnt >= K, cand, t)

        t = lax.fori_loop(0, 32, body, jnp.int32(INT_MIN))
        thr_ref[...] = jnp.full((1, 16), t, jnp.int32)


def _k1(node_embs, norm2d, scorer):
    return pl.pallas_call(
        _k1_body,
        grid=(GRID,),
        in_specs=[
            pl.BlockSpec((1, 1), lambda i: (0, 0)),
            pl.BlockSpec((BLK, F), lambda i: (i, 0)),
            pl.BlockSpec((F, 1), lambda i: (0, 0)),
        ],
        out_specs=[
            pl.BlockSpec((1, 1, BLK), lambda i: (i, 0, 0)),
            pl.BlockSpec((1, 16), lambda i: (0, 0)),
        ],
        out_shape=[
            jax.ShapeDtypeStruct((GRID, 1, BLK), jnp.int32),
            jax.ShapeDtypeStruct((1, 16), jnp.int32),
        ],
        scratch_shapes=[pltpu.VMEM((GRID, BLK), jnp.int32)],
    )(norm2d, node_embs, scorer)


def _k2_body(keys_hbm, thr_hbm, ck_hbm, ci_hbm, keys_v, thr_v, lk, li):
    w = lax.axis_index("s") * 2 + lax.axis_index("c")
    base = w * CHUNK
    pltpu.sync_copy(keys_hbm.at[pl.ds(base, CHUNK)], keys_v)
    pltpu.sync_copy(thr_hbm, thr_v)
    thrv = thr_v[...]
    for i in range((CAP + 16) // 16):
        lk[pl.ds(i * 16, 16)] = jnp.full((16,), INT_MIN, jnp.int32)
        li[pl.ds(i * 16, 16)] = jnp.zeros((16,), jnp.int32)

    def body(i, off):
        kv = keys_v[pl.ds(i * 16, 16)]
        m = kv >= thrv
        iv = base + i * 16 + lax.iota(jnp.int32, 16)
        mi = jnp.where(m, 1, 0)
        c = plsc.cumsum(mi)
        pos = (off + c) - 1
        plsc.store_scatter(lk, [pos], kv, mask=m)
        plsc.store_scatter(li, [pos], iv, mask=m)
        cnt = jnp.sum(mi, axis=0)
        return jnp.minimum(off + cnt, CHUNK)

    lax.fori_loop(0, CHUNK // 16, body, jnp.int32(0))
    pltpu.sync_copy(lk.at[pl.ds(0, CAP)], ck_hbm.at[w])
    pltpu.sync_copy(li.at[pl.ds(0, CAP)], ci_hbm.at[w])


def _k2(keys_flat, thr16):
    mesh = plsc.VectorSubcoreMesh(core_axis_name="c", subcore_axis_name="s")
    fn = pl.kernel(
        _k2_body,
        mesh=mesh,
        compiler_params=pltpu.CompilerParams(needs_layout_passes=False),
        out_type=[
            jax.ShapeDtypeStruct((NW, CAP), jnp.int32),
            jax.ShapeDtypeStruct((NW, CAP), jnp.int32),
        ],
        scratch_types=[
            pltpu.VMEM((CHUNK,), jnp.int32),
            pltpu.VMEM((16,), jnp.int32),
            pltpu.VMEM((CHUNK + 16,), jnp.int32),
            pltpu.VMEM((CHUNK + 16,), jnp.int32),
        ],
    )
    return fn(keys_flat, thr16)


def _xor_shuffle(x, j):
    if j < 128:
        c = lax.broadcasted_iota(jnp.int32, x.shape, 1)
        return jnp.where((c & j) == 0,
                         jnp.roll(x, -j, axis=1), jnp.roll(x, j, axis=1))
    jr = j // 128
    r = lax.broadcasted_iota(jnp.int32, x.shape, 0)
    return jnp.where((r & jr) == 0,
                     jnp.roll(x, -jr, axis=0), jnp.roll(x, jr, axis=0))


def _k3_body(ck_ref, ci_ref, keys_out, gate_out):
    kk = ck_ref[...]
    ii = ci_ref[...]
    n = kk.shape[0] * kk.shape[1]
    r = lax.broadcasted_iota(jnp.int32, kk.shape, 0)
    c = lax.broadcasted_iota(jnp.int32, kk.shape, 1)
    p = r * 128 + c
    ksize = 2
    while ksize <= n:
        j = ksize // 2
        while j >= 1:
            pk = _xor_shuffle(kk, j)
            pi = _xor_shuffle(ii, j)
            is_lower = (p & j) == 0
            dirn_desc = (p & ksize) == 0
            a_first = (kk > pk) | ((kk == pk) & (ii < pi))
            take_larger = is_lower == dirn_desc
            sel = take_larger == a_first
            kk = jnp.where(sel, kk, pk)
            ii = jnp.where(sel, ii, pi)
            j //= 2
        ksize *= 2
    top_k = kk[:K // 128]
    top_i = ii[:K // 128]
    bits = jnp.where(top_k < 0, top_k ^ MASK31, top_k)
    s = lax.bitcast_convert_type(bits, jnp.float32)
    keys_out[...] = top_i
    gate_out[...] = jnp.tanh(s)


def _k3(ck, ci):
    return pl.pallas_call(
        _k3_body,
        out_shape=[
            jax.ShapeDtypeStruct((K // 128, 128), jnp.int32),
            jax.ShapeDtypeStruct((K // 128, 128), jnp.float32),
        ],
    )(ck, ci)


def _k4_body(embs_hbm, idx_hbm, gate_hbm, out_hbm, idx_v, g_v, rows_v, sem):
    w = lax.axis_index("s") * 2 + lax.axis_index("c")
    base = w * (K // NW)
    pltpu.sync_copy(idx_hbm.at[pl.ds(base, K // NW)], idx_v)
    pltpu.sync_copy(gate_hbm.at[pl.ds(base, K // NW)], g_v)
    pltpu.async_copy(embs_hbm.at[idx_v], rows_v, sem).wait()
    for r in range(K // NW):
        g = g_v[pl.ds((r // 16) * 16, 16)][r % 16]
        for cc in range(F // 16):
            sl = pl.ds(cc * 16, 16)
            rows_v[r, sl] = rows_v[r, sl] * g
    pltpu.sync_copy(rows_v, out_hbm.at[pl.ds(base, K // NW)])


def _k4(node_embs, idx_flat, gate_flat):
    mesh = plsc.VectorSubcoreMesh(core_axis_name="c", subcore_axis_name="s")
    fn = pl.kernel(
        _k4_body,
        mesh=mesh,
        compiler_params=pltpu.CompilerParams(needs_layout_passes=False),
        out_type=jax.ShapeDtypeStruct((K, F), jnp.float32),
        scratch_types=[
            pltpu.VMEM((K // NW,), jnp.int32),
            pltpu.VMEM((K // NW,), jnp.float32),
            pltpu.VMEM((K // NW, F), jnp.float32),
            pltpu.SemaphoreType.DMA,
        ],
    )
    return fn(node_embs, idx_flat, gate_flat)


def kernel(node_embs, scorer):
    norm2d = jnp.linalg.norm(scorer).reshape(1, 1)
    keys3, thr = _k1(node_embs, norm2d, scorer)
    ck, ci = _k2(keys3.reshape(-1), thr.reshape(16))
    idx8, gate8 = _k3(ck, ci)
    rows = _k4(node_embs, idx8.reshape(-1), gate8.reshape(-1))
    return rows.T


# speedup vs baseline: 3.5611x; 1.0680x over previous
import functools

import jax
import jax.numpy as jnp
from jax import lax
from jax.experimental import pallas as pl
from jax.experimental.pallas import tpu as pltpu
from jax.experimental.pallas import tpu_sc as plsc

N = 100000
F = 128
K = 1024
BLK = 12544
GRID = 8
NPAD = BLK * GRID
NW = 32
CHUNK = NPAD // NW
CAP = 128
INT_MIN = -2147483648
MASK31 = 0x7FFFFFFF


def _k1_body(norm_ref, x_ref, w_ref, keys_ref, thr_ref, skeys_ref):
    b = pl.program_id(0)
    s = lax.dot_general(
        w_ref[...].reshape(1, F), x_ref[...],
        dimension_numbers=(((1,), (1,)), ((), ())),
        preferred_element_type=jnp.float32)
    s = s / norm_ref[0, 0]
    k = lax.bitcast_convert_type(s, jnp.int32)
    k = jnp.where(k < 0, k ^ MASK31, k)
    pos = b * BLK + lax.broadcasted_iota(jnp.int32, (1, BLK), 1)
    k = jnp.where(pos < N, k, INT_MIN)
    keys_ref[...] = k[None]
    skeys_ref[pl.ds(b, 1), :] = k

    @pl.when(b == GRID - 1)
    def _():
        allk = skeys_ref[...]

        def body(i, t):
            cand = t + (jnp.int32(1) << (31 - i))
            cnt = jnp.sum((allk >= cand).astype(jnp.int32))
            return jnp.where(cnt >= K, cand, t)

        t = lax.fori_loop(0, 32, body, jnp.int32(INT_MIN))
        thr_ref[...] = jnp.full((1, 16), t, jnp.int32)


def _k1(node_embs, norm2d, scorer):
    return pl.pallas_call(
        _k1_body,
        grid=(GRID,),
        in_specs=[
            pl.BlockSpec((1, 1), lambda i: (0, 0)),
            pl.BlockSpec((BLK, F), lambda i: (i, 0)),
            pl.BlockSpec((F, 1), lambda i: (0, 0)),
        ],
        out_specs=[
            pl.BlockSpec((1, 1, BLK), lambda i: (i, 0, 0)),
            pl.BlockSpec((1, 16), lambda i: (0, 0)),
        ],
        out_shape=[
            jax.ShapeDtypeStruct((GRID, 1, BLK), jnp.int32),
            jax.ShapeDtypeStruct((1, 16), jnp.int32),
        ],
        scratch_shapes=[pltpu.VMEM((GRID, BLK), jnp.int32)],
    )(norm2d, node_embs, scorer)


def _k2_body(keys_hbm, thr_hbm, ck_hbm, ci_hbm, keys_v, thr_v, lk, li):
    w = lax.axis_index("s") * 2 + lax.axis_index("c")
    base = w * CHUNK
    pltpu.sync_copy(keys_hbm.at[pl.ds(base, CHUNK)], keys_v)
    pltpu.sync_copy(thr_hbm, thr_v)
    thrv = thr_v[...]
    for i in range((CAP + 16) // 16):
        lk[pl.ds(i * 16, 16)] = jnp.full((16,), INT_MIN, jnp.int32)
        li[pl.ds(i * 16, 16)] = jnp.zeros((16,), jnp.int32)

    def body(i, off):
        kv = keys_v[pl.ds(i * 16, 16)]
        m = kv >= thrv
        iv = base + i * 16 + lax.iota(jnp.int32, 16)
        mi = jnp.where(m, 1, 0)
        c = plsc.cumsum(mi)
        pos = (off + c) - 1
        plsc.store_scatter(lk, [pos], kv, mask=m)
        plsc.store_scatter(li, [pos], iv, mask=m)
        cnt = jnp.sum(mi, axis=0)
        return jnp.minimum(off + cnt, CHUNK)

    lax.fori_loop(0, CHUNK // 16, body, jnp.int32(0))
    pltpu.sync_copy(lk.at[pl.ds(0, CAP)], ck_hbm.at[w])
    pltpu.sync_copy(li.at[pl.ds(0, CAP)], ci_hbm.at[w])


def _k2(keys_flat, thr16):
    mesh = plsc.VectorSubcoreMesh(core_axis_name="c", subcore_axis_name="s")
    fn = pl.kernel(
        _k2_body,
        mesh=mesh,
        compiler_params=pltpu.CompilerParams(needs_layout_passes=False),
        out_type=[
            jax.ShapeDtypeStruct((NW, CAP), jnp.int32),
            jax.ShapeDtypeStruct((NW, CAP), jnp.int32),
        ],
        scratch_types=[
            pltpu.VMEM((CHUNK,), jnp.int32),
            pltpu.VMEM((16,), jnp.int32),
            pltpu.VMEM((CHUNK + 16,), jnp.int32),
            pltpu.VMEM((CHUNK + 16,), jnp.int32),
        ],
    )
    return fn(keys_flat, thr16)


def _xor_shuffle(x, j):
    if j < 128:
        c = lax.broadcasted_iota(jnp.int32, x.shape, 1)
        return jnp.where((c & j) == 0,
                         jnp.roll(x, -j, axis=1), jnp.roll(x, j, axis=1))
    jr = j // 128
    r = lax.broadcasted_iota(jnp.int32, x.shape, 0)
    return jnp.where((r & jr) == 0,
                     jnp.roll(x, -jr, axis=0), jnp.roll(x, jr, axis=0))


def _k3_body(ck_ref, ci_ref, keys_out, gate_out):
    kk = ck_ref[...]
    ii = ci_ref[...]
    n = kk.shape[0] * kk.shape[1]
    r = lax.broadcasted_iota(jnp.int32, kk.shape, 0)
    c = lax.broadcasted_iota(jnp.int32, kk.shape, 1)
    p = r * 128 + c
    ksize = 2
    while ksize <= n:
        j = ksize // 2
        while j >= 1:
            pk = _xor_shuffle(kk, j)
            pi = _xor_shuffle(ii, j)
            is_lower = (p & j) == 0
            dirn_desc = (p & ksize) == 0
            a_first = (kk > pk) | ((kk == pk) & (ii < pi))
            take_larger = is_lower == dirn_desc
            sel = take_larger == a_first
            kk = jnp.where(sel, kk, pk)
            ii = jnp.where(sel, ii, pi)
            j //= 2
        ksize *= 2
    top_k = kk[:K // 128]
    top_i = ii[:K // 128]
    bits = jnp.where(top_k < 0, top_k ^ MASK31, top_k)
    s = lax.bitcast_convert_type(bits, jnp.float32)
    keys_out[...] = top_i
    gate_out[...] = jnp.tanh(s)


def _k3(ck, ci):
    return pl.pallas_call(
        _k3_body,
        out_shape=[
            jax.ShapeDtypeStruct((K // 128, 128), jnp.int32),
            jax.ShapeDtypeStruct((K // 128, 128), jnp.float32),
        ],
    )(ck, ci)


RPT = K // NW


def _k4_body(embs_hbm, idx_hbm, gate_hbm, out_hbm, idx_v, g_v, rows_v, t_v, sem):
    w = lax.axis_index("s") * 2 + lax.axis_index("c")
    base = w * RPT
    pltpu.sync_copy(idx_hbm.at[pl.ds(base, RPT)], idx_v)
    pltpu.sync_copy(gate_hbm.at[pl.ds(base, RPT)], g_v)
    pltpu.async_copy(embs_hbm.at[idx_v], rows_v, sem).wait()
    for r in range(RPT):
        g = g_v[pl.ds((r // 16) * 16, 16)][r % 16]
        for cc in range(F // 16):
            sl = pl.ds(cc * 16, 16)
            t_v[r, sl] = rows_v[r, sl] * g
    pltpu.sync_copy(t_v, out_hbm.at[pl.ds(base, RPT)])


def _k4(node_embs, idx_flat, gate_flat):
    mesh = plsc.VectorSubcoreMesh(core_axis_name="c", subcore_axis_name="s")
    fn = pl.kernel(
        _k4_body,
        mesh=mesh,
        compiler_params=pltpu.CompilerParams(needs_layout_passes=False),
        out_type=jax.ShapeDtypeStruct((K, F), jnp.float32),
        scratch_types=[
            pltpu.VMEM((RPT,), jnp.int32),
            pltpu.VMEM((RPT,), jnp.float32),
            pltpu.VMEM((RPT, F), jnp.float32),
            pltpu.VMEM((RPT, F), jnp.float32),
            pltpu.SemaphoreType.DMA,
        ],
    )
    return fn(node_embs, idx_flat, gate_flat)


def kernel(node_embs, scorer):
    norm2d = jnp.linalg.norm(scorer).reshape(1, 1)
    keys3, thr = _k1(node_embs, norm2d, scorer)
    ck, ci = _k2(keys3.reshape(-1), thr.reshape(16))
    idx8, gate8 = _k3(ck, ci)
    rows = _k4(node_embs, idx8.reshape(-1), gate8.reshape(-1))
    return rows.T
